# baseline (speedup 1.0000x reference)
import functools

import jax
import jax.numpy as jnp
from jax.experimental import pallas as pl
from jax.experimental.pallas import tpu as pltpu
from jax.experimental.pallas import tpu_sc as plsc

S, N, D = 1024, 8, 256
M = S * N
K = 8192
BM = 256


def _vq_block_kernel(x_ref, cb_ref, xsq_ref, csq_ref, aux_ref,
                     loss_ref, oh_ref, idx_ref):
    i = pl.program_id(0)
    x = x_ref[...]
    cb = cb_ref[...]
    mm = jax.lax.dot_general(x, cb, (((1,), (1,)), ((), ())),
                             preferred_element_type=jnp.float32)
    d = (xsq_ref[...] + csq_ref[...]) - 2.0 * mm
    dmin = jnp.min(d, axis=1, keepdims=True)
    ohm = (d == dmin).astype(jnp.float32)
    oh_ref[...] = ohm
    agg = jax.lax.dot_general(ohm, aux_ref[...], (((1,), (0,)), ((), ())),
                              precision=jax.lax.Precision.HIGHEST,
                              preferred_element_type=jnp.float32)
    cnt = agg[:, 1:2]
    tied = jnp.max(cnt) > 1.5

    @pl.when(jnp.logical_not(tied))
    def _fast():
        idx_ref[...] = (agg[:, 0:1] + 0.5).astype(jnp.int32)

    @pl.when(tied)
    def _slow():
        kio = jax.lax.broadcasted_iota(jnp.int32, d.shape, 1)
        idx = jnp.min(jnp.where(d == dmin, kio, K), axis=1, keepdims=True)
        idx_ref[...] = idx
        oh_ref[...] = (kio == idx).astype(jnp.float32)

    @pl.when(i == 0)
    def _init():
        loss_ref[...] = jnp.zeros_like(loss_ref)

    loss_ref[...] += jnp.sum(dmin).reshape(1, 1)


@functools.cache
def _sc_gather_fn():
    info = plsc.get_sparse_core_info()
    nc = info.num_cores
    nw = nc * info.num_subcores
    bpw = M // nw

    def _body(table_hbm, idx_hbm, out_hbm, idx_v, rows_v, sem):
        wid = jax.lax.axis_index("s") * nc + jax.lax.axis_index("c")
        base = wid * bpw
        pltpu.sync_copy(idx_hbm.at[pl.ds(base, bpw)], idx_v)
        pltpu.async_copy(table_hbm.at[idx_v], rows_v, sem).wait()
        pltpu.sync_copy(rows_v, out_hbm.at[pl.ds(base, bpw)])

    return functools.partial(
        pl.kernel,
        mesh=plsc.VectorSubcoreMesh(core_axis_name="c", subcore_axis_name="s"),
        out_type=jax.ShapeDtypeStruct((M, D), jnp.float32),
        scratch_types=[
            pltpu.VMEM((bpw,), jnp.int32),
            pltpu.VMEM((bpw, D), jnp.float32),
            pltpu.SemaphoreType.DMA,
        ],
    )(_body)


@jax.jit
def kernel(inputs, codebook):
    flat = inputs.reshape(-1, D)
    xsq = jnp.sum(flat ** 2, axis=1, keepdims=True)
    csq = jnp.sum(codebook ** 2, axis=1)[None, :]
    aux = jnp.stack([jnp.arange(K, dtype=jnp.float32),
                     jnp.ones((K,), jnp.float32)], axis=1)

    grid = (M // BM,)
    loss_acc, oh, idx = pl.pallas_call(
        _vq_block_kernel,
        grid=grid,
        in_specs=[
            pl.BlockSpec((BM, D), lambda i: (i, 0)),
            pl.BlockSpec((K, D), lambda i: (0, 0)),
            pl.BlockSpec((BM, 1), lambda i: (i, 0)),
            pl.BlockSpec((1, K), lambda i: (0, 0)),
            pl.BlockSpec((K, 2), lambda i: (0, 0)),
        ],
        out_specs=[
            pl.BlockSpec((1, 1), lambda i: (0, 0)),
            pl.BlockSpec((BM, K), lambda i: (i, 0)),
            pl.BlockSpec((BM, 1), lambda i: (i, 0)),
        ],
        out_shape=[
            jax.ShapeDtypeStruct((1, 1), jnp.float32),
            jax.ShapeDtypeStruct((M, K), jnp.float32),
            jax.ShapeDtypeStruct((M, 1), jnp.int32),
        ],
    )(flat, codebook, xsq, csq, aux)

    q = _sc_gather_fn()(codebook, idx.reshape(M))

    loss = loss_acc[0, 0] * (0.25 / (M * D))
    quantized_st = q.reshape(S, N, D)
    encodings_flat = oh.reshape(S, N, K)
    return (loss, quantized_st, encodings_flat, idx)


# speedup vs baseline: 1.6475x; 1.6475x over previous
import functools

import jax
import jax.numpy as jnp
from jax.experimental import pallas as pl
from jax.experimental.pallas import tpu as pltpu
from jax.experimental.pallas import tpu_sc as plsc

S, N, D = 1024, 8, 256
M = S * N
K = 8192
BM = 256


def _vq_block_kernel(x_ref, cb_ref, xsq_ref, csq_ref, aux_ref,
                     loss_ref, oh_ref, idx_ref):
    i = pl.program_id(0)
    x = x_ref[...]
    cb = cb_ref[...]
    mm = jax.lax.dot_general(x, cb, (((1,), (1,)), ((), ())),
                             preferred_element_type=jnp.float32)
    d = (xsq_ref[...] + csq_ref[...]) - 2.0 * mm
    dmin = jnp.min(d, axis=1, keepdims=True)
    ohm = (d == dmin).astype(jnp.float32)
    oh_ref[...] = ohm
    agg = jax.lax.dot_general(ohm.astype(jnp.bfloat16), aux_ref[...],
                              (((1,), (1,)), ((), ())),
                              preferred_element_type=jnp.float32)
    cnt = agg[:, 2:3]
    tied = jnp.max(cnt) > 1.5

    @pl.when(jnp.logical_not(tied))
    def _fast():
        idx_ref[...] = (256.0 * agg[:, 0:1] + agg[:, 1:2]
                        + 0.5).astype(jnp.int32)

    @pl.when(tied)
    def _slow():
        kio = jax.lax.broadcasted_iota(jnp.int32, d.shape, 1)
        idx = jnp.min(jnp.where(d == dmin, kio, K), axis=1, keepdims=True)
        idx_ref[...] = idx
        oh_ref[...] = (kio == idx).astype(jnp.float32)

    @pl.when(i == 0)
    def _init():
        loss_ref[...] = jnp.zeros_like(loss_ref)

    loss_ref[...] += jnp.sum(dmin).reshape(1, 1)


@functools.cache
def _sc_gather_fn():
    info = plsc.get_sparse_core_info()
    nc = info.num_cores
    nw = nc * info.num_subcores
    bpw = M // nw

    def _body(table_hbm, idx_hbm, out_hbm, idx_v, rows_v, sem):
        wid = jax.lax.axis_index("s") * nc + jax.lax.axis_index("c")
        base = wid * bpw
        pltpu.sync_copy(idx_hbm.at[pl.ds(base, bpw)], idx_v)
        pltpu.async_copy(table_hbm.at[idx_v], rows_v, sem).wait()
        pltpu.sync_copy(rows_v, out_hbm.at[pl.ds(base, bpw)])

    return functools.partial(
        pl.kernel,
        mesh=plsc.VectorSubcoreMesh(core_axis_name="c", subcore_axis_name="s"),
        out_type=jax.ShapeDtypeStruct((M, D), jnp.float32),
        scratch_types=[
            pltpu.VMEM((bpw,), jnp.int32),
            pltpu.VMEM((bpw, D), jnp.float32),
            pltpu.SemaphoreType.DMA,
        ],
    )(_body)


@jax.jit
def kernel(inputs, codebook):
    flat = inputs.reshape(-1, D)
    xsq = jnp.sum(flat ** 2, axis=1, keepdims=True)
    csq = jnp.sum(codebook ** 2, axis=1)[None, :]
    kio = jnp.arange(K, dtype=jnp.int32)
    aux = jnp.zeros((8, K), jnp.bfloat16)
    aux = aux.at[0].set((kio >> 8).astype(jnp.bfloat16))
    aux = aux.at[1].set((kio & 255).astype(jnp.bfloat16))
    aux = aux.at[2].set(jnp.ones((K,), jnp.bfloat16))

    grid = (M // BM,)
    loss_acc, oh, idx = pl.pallas_call(
        _vq_block_kernel,
        grid=grid,
        in_specs=[
            pl.BlockSpec((BM, D), lambda i: (i, 0)),
            pl.BlockSpec((K, D), lambda i: (0, 0)),
            pl.BlockSpec((BM, 1), lambda i: (i, 0)),
            pl.BlockSpec((1, K), lambda i: (0, 0)),
            pl.BlockSpec((8, K), lambda i: (0, 0)),
        ],
        out_specs=[
            pl.BlockSpec((1, 1), lambda i: (0, 0)),
            pl.BlockSpec((BM, K), lambda i: (i, 0)),
            pl.BlockSpec((BM, 1), lambda i: (i, 0)),
        ],
        out_shape=[
            jax.ShapeDtypeStruct((1, 1), jnp.float32),
            jax.ShapeDtypeStruct((M, K), jnp.float32),
            jax.ShapeDtypeStruct((M, 1), jnp.int32),
        ],
    )(flat, codebook, xsq, csq, aux)

    q = _sc_gather_fn()(codebook, idx.reshape(M))

    loss = loss_acc[0, 0] * (0.25 / (M * D))
    quantized_st = q.reshape(S, N, D)
    encodings_flat = oh.reshape(S, N, K)
    return (loss, quantized_st, encodings_flat, idx)


# speedup vs baseline: 2.3821x; 1.4459x over previous
import functools

import jax
import jax.numpy as jnp
from jax.experimental import pallas as pl
from jax.experimental.pallas import tpu as pltpu
from jax.experimental.pallas import tpu_sc as plsc

S, N, D = 1024, 8, 256
M = S * N
K = 8192
BM = 256


def _vq_block_kernel(x_ref, cb2_ref, xsq_ref, csq_ref, kiof_ref,
                     loss_ref, oh_ref, idx_ref):
    i = pl.program_id(0)
    x = x_ref[...]
    cb2 = cb2_ref[...]
    mm2 = jax.lax.dot_general(x, cb2, (((1,), (1,)), ((), ())),
                              preferred_element_type=jnp.float32)
    d = (xsq_ref[...] + csq_ref[...]) + mm2
    dmin = jnp.min(d, axis=1, keepdims=True)
    tmp = jnp.where(d == dmin, kiof_ref[...], jnp.inf)
    idx_f = jnp.min(tmp, axis=1, keepdims=True)
    idx_ref[...] = idx_f.astype(jnp.int32)
    oh_ref[...] = (tmp == idx_f).astype(jnp.float32)

    @pl.when(i == 0)
    def _init():
        loss_ref[...] = jnp.zeros_like(loss_ref)

    loss_ref[...] += jnp.sum(dmin).reshape(1, 1)


@functools.cache
def _sc_gather_fn():
    info = plsc.get_sparse_core_info()
    nc = info.num_cores
    nw = nc * info.num_subcores
    bpw = M // nw

    def _body(table_hbm, idx_hbm, out_hbm, idx_v, rows_v, sem):
        wid = jax.lax.axis_index("s") * nc + jax.lax.axis_index("c")
        base = wid * bpw
        pltpu.sync_copy(idx_hbm.at[pl.ds(base, bpw)], idx_v)
        pltpu.async_copy(table_hbm.at[idx_v], rows_v, sem).wait()
        pltpu.sync_copy(rows_v, out_hbm.at[pl.ds(base, bpw)])

    return functools.partial(
        pl.kernel,
        mesh=plsc.VectorSubcoreMesh(core_axis_name="c", subcore_axis_name="s"),
        out_type=jax.ShapeDtypeStruct((M, D), jnp.float32),
        scratch_types=[
            pltpu.VMEM((bpw,), jnp.int32),
            pltpu.VMEM((bpw, D), jnp.float32),
            pltpu.SemaphoreType.DMA,
        ],
    )(_body)


@jax.jit
def kernel(inputs, codebook):
    flat = inputs.reshape(-1, D)
    xsq = jnp.sum(flat ** 2, axis=1, keepdims=True)
    csq = jnp.sum(codebook ** 2, axis=1)[None, :]
    kiof = jnp.arange(K, dtype=jnp.float32)[None, :]
    cb2 = -2.0 * codebook

    grid = (M // BM,)
    loss_acc, oh, idx = pl.pallas_call(
        _vq_block_kernel,
        grid=grid,
        in_specs=[
            pl.BlockSpec((BM, D), lambda i: (i, 0)),
            pl.BlockSpec((K, D), lambda i: (0, 0)),
            pl.BlockSpec((BM, 1), lambda i: (i, 0)),
            pl.BlockSpec((1, K), lambda i: (0, 0)),
            pl.BlockSpec((1, K), lambda i: (0, 0)),
        ],
        out_specs=[
            pl.BlockSpec((1, 1), lambda i: (0, 0)),
            pl.BlockSpec((BM, K), lambda i: (i, 0)),
            pl.BlockSpec((BM, 1), lambda i: (i, 0)),
        ],
        out_shape=[
            jax.ShapeDtypeStruct((1, 1), jnp.float32),
            jax.ShapeDtypeStruct((M, K), jnp.float32),
            jax.ShapeDtypeStruct((M, 1), jnp.int32),
        ],
    )(flat, cb2, xsq, csq, kiof)

    q = _sc_gather_fn()(codebook, idx.reshape(M))

    loss = loss_acc[0, 0] * (0.25 / (M * D))
    quantized_st = q.reshape(S, N, D)
    encodings_flat = oh.reshape(S, N, K)
    return (loss, quantized_st, encodings_flat, idx)
